# Initial kernel scaffold; baseline (speedup 1.0000x reference)
import jax
import jax.numpy as jnp
from jax.experimental import pallas as pl


def kernel(X, W, b):
    raise NotImplementedError("write your pallas kernel here")


# speedup vs baseline: 1.4157x; 1.4157x over previous
import numpy as np
import jax
import jax.numpy as jnp
from jax.experimental import pallas as pl
from jax.experimental.pallas import tpu as pltpu

_B = 1024
_H = 32768
_C = 3
_R = 8
_K = 256
_L = 128
_W3 = _K * _L * _C // _K
_STARTS = (0, 8768, 16768, 24768)
_ENDS = (8768, 16768, 24768, 32768)
_TINY = np.float32(np.finfo(np.float32).tiny)
_NEG_INF = np.float32(-np.inf)


def _threefry_bits(k0, k1, x1):
    rot_a = (13, 15, 26, 6)
    rot_b = (17, 29, 16, 24)
    ks2 = k0 ^ k1 ^ np.uint32(0x1BD11BDA)
    x0 = jnp.broadcast_to(k0, x1.shape)
    x1 = x1 + k1

    def rounds(x0, x1, rots):
        for r in rots:
            x0 = x0 + x1
            x1 = (x1 << np.uint32(r)) | (x1 >> np.uint32(32 - r))
            x1 = x0 ^ x1
        return x0, x1

    x0, x1 = rounds(x0, x1, rot_a)
    x0, x1 = x0 + k1, x1 + ks2 + np.uint32(1)
    x0, x1 = rounds(x0, x1, rot_b)
    x0, x1 = x0 + ks2, x1 + k0 + np.uint32(2)
    x0, x1 = rounds(x0, x1, rot_a)
    x0, x1 = x0 + k0, x1 + k1 + np.uint32(3)
    x0, x1 = rounds(x0, x1, rot_b)
    x0, x1 = x0 + k1, x1 + ks2 + np.uint32(4)
    x0, x1 = rounds(x0, x1, rot_a)
    x0, x1 = x0 + ks2, x1 + k0 + np.uint32(5)
    return x0 ^ x1


def _gumbel(k0, k1, idx):
    bits = _threefry_bits(k0, k1, idx)
    fb = (bits >> np.uint32(9)) | np.uint32(0x3F800000)
    floats = jax.lax.bitcast_convert_type(fb, jnp.float32) - np.float32(1.0)
    u = jnp.maximum(_TINY, floats * (np.float32(1.0) - _TINY) + _TINY)
    return -jnp.log(-jnp.log(u))


def _first_argmax(v, hmat):
    m = jnp.max(v, axis=(1, 2), keepdims=True)
    sel = jnp.where(v == m, hmat, jnp.int32(_H))
    return jnp.min(sel, axis=(1, 2))


def _body(key_ref, x_ref, w_ref, bias_ref, comb_ref, out_ref):
    i = pl.program_id(0)
    xb = x_ref[...].astype(jnp.bfloat16)
    sel = jax.lax.dot_general(
        xb, comb_ref[...],
        dimension_numbers=(((2,), (0,)), ((), ())),
        preferred_element_type=jnp.float32)
    l0 = sel[:, :, 0:_L]
    l1 = sel[:, :, _L:2 * _L]
    l2 = sel[:, :, 2 * _L:3 * _L]
    w_f = w_ref[...].astype(jnp.float32)
    p0 = l0 * w_f[0][None]
    p1 = l1 * w_f[1][None]
    p2 = l2 * w_f[2][None]
    sum_a = (p0 + p2) + p1
    sum_b = (p1 + p2) + p0

    hmat = (jax.lax.broadcasted_iota(jnp.int32, (_R, _K, _L), 1) * _L
            + jax.lax.broadcasted_iota(jnp.int32, (_R, _K, _L), 2))
    masks = [(hmat >= s) & (hmat < e) for s, e in zip(_STARTS, _ENDS)]
    logits = jnp.where(masks[2], sum_b, sum_a) + bias_ref[...]

    mseg = [jnp.max(jnp.where(mk, logits, _NEG_INF), axis=(1, 2), keepdims=True)
            for mk in masks]
    mfull = jnp.where(masks[0], mseg[0],
                      jnp.where(masks[1], mseg[1],
                                jnp.where(masks[2], mseg[2], mseg[3])))
    ez = jnp.exp(logits - mfull)
    sseg = [jnp.sum(jnp.where(mk, ez, np.float32(0.0)), axis=(1, 2), keepdims=True)
            for mk in masks]
    sfull = jnp.where(masks[0], sseg[0],
                      jnp.where(masks[1], sseg[1],
                                jnp.where(masks[2], sseg[2], sseg[3])))
    p = ez / sfull
    logp = jnp.log(p)
    best = _first_argmax(p, hmat)

    k0 = key_ref[0].astype(jnp.uint32)
    k1 = key_ref[1].astype(jnp.uint32)
    row = (i * _R + jax.lax.broadcasted_iota(jnp.int32, (_R, _K, _L), 0))
    base = row * _H + hmat
    samples = []
    for e in range(3):
        idx = (base + np.int32(e * _B * _H)).astype(jnp.uint32)
        v = logp + _gumbel(k0, k1, idx)
        samples.append(_first_argmax(v, hmat))
    s0, s1, s2 = samples
    chosen = jnp.where(s0 == best, s0, jnp.where(s1 == best, s1, s2))
    out_ref[0, 0, :] = chosen


def kernel(X, W, b):
    X3 = X.reshape(_B, _K, _C * _L)

    wbf = W.astype(jnp.bfloat16)
    wcols = []
    for c in range(_C):
        parts = [jnp.broadcast_to(wbf[s, c, 0], (_ENDS[s] - _STARTS[s],))
                 for s in range(4)]
        wcols.append(jnp.concatenate(parts))
    wfull = jnp.stack(wcols).reshape(_C, _K, _L)
    bparts = [jnp.broadcast_to(b[s, 0], (_ENDS[s] - _STARTS[s],)) for s in range(4)]
    bfull = jnp.concatenate(bparts).reshape(1, _K, _L)

    j = np.arange(_C * _L)[:, None]
    cd = np.arange(_C * _L)[None, :]
    comb = jnp.asarray(j == 3 * (cd % _L) + cd // _L, dtype=jnp.bfloat16)

    kd = jax.random.key_data(jax.random.split(jax.random.key(1))[1])
    kd = kd.astype(jnp.int32)

    grid = _B // _R
    out = pl.pallas_call(
        _body,
        grid=grid,
        in_specs=[
            pl.BlockSpec(memory_space=pltpu.SMEM),
            pl.BlockSpec((_R, _K, _C * _L), lambda i: (i, 0, 0)),
            pl.BlockSpec((_C, _K, _L), lambda i: (0, 0, 0)),
            pl.BlockSpec((1, _K, _L), lambda i: (0, 0, 0)),
            pl.BlockSpec((_C * _L, _C * _L), lambda i: (0, 0)),
        ],
        out_specs=pl.BlockSpec((1, 1, _R), lambda i: (i, 0, 0)),
        out_shape=jax.ShapeDtypeStruct((grid, 1, _R), jnp.int32),
        compiler_params=pltpu.CompilerParams(
            dimension_semantics=("parallel",)),
    )(kd, X3, wfull, bfull, comb)
    return out.reshape(_B)


# speedup vs baseline: 1.6638x; 1.1752x over previous
import numpy as np
import jax
import jax.numpy as jnp
from jax.experimental import pallas as pl
from jax.experimental.pallas import tpu as pltpu

_B = 1024
_H = 32768
_C = 3
_R = 8
_K = 256
_L = 128
_W3 = _K * _L * _C // _K
_STARTS = (0, 8768, 16768, 24768)
_ENDS = (8768, 16768, 24768, 32768)
_TINY = np.float32(np.finfo(np.float32).tiny)
_NEG_INF = np.float32(-np.inf)


def _threefry_bits(k0, k1, x1):
    rot_a = (13, 15, 26, 6)
    rot_b = (17, 29, 16, 24)
    ks2 = k0 ^ k1 ^ np.uint32(0x1BD11BDA)
    x0 = jnp.broadcast_to(k0, x1.shape)
    x1 = x1 + k1

    def rounds(x0, x1, rots):
        for r in rots:
            x0 = x0 + x1
            x1 = (x1 << np.uint32(r)) | (x1 >> np.uint32(32 - r))
            x1 = x0 ^ x1
        return x0, x1

    x0, x1 = rounds(x0, x1, rot_a)
    x0, x1 = x0 + k1, x1 + ks2 + np.uint32(1)
    x0, x1 = rounds(x0, x1, rot_b)
    x0, x1 = x0 + ks2, x1 + k0 + np.uint32(2)
    x0, x1 = rounds(x0, x1, rot_a)
    x0, x1 = x0 + k0, x1 + k1 + np.uint32(3)
    x0, x1 = rounds(x0, x1, rot_b)
    x0, x1 = x0 + k1, x1 + ks2 + np.uint32(4)
    x0, x1 = rounds(x0, x1, rot_a)
    x0, x1 = x0 + ks2, x1 + k0 + np.uint32(5)
    return x0 ^ x1


def _gumbel(k0, k1, idx):
    bits = _threefry_bits(k0, k1, idx)
    fb = (bits >> np.uint32(9)) | np.uint32(0x3F800000)
    floats = jax.lax.bitcast_convert_type(fb, jnp.float32) - np.float32(1.0)
    u = jnp.maximum(_TINY, floats + _TINY)
    return -jnp.log(-jnp.log(u))


def _first_argmax(v, hmat):
    m = jnp.max(v, axis=(1, 2), keepdims=True)
    sel = jnp.where(v == m, hmat, jnp.int32(_H))
    return jnp.min(sel, axis=(1, 2))


def _body(key_ref, x_ref, w_ref, bias_ref, comb_ref, out_ref):
    i = pl.program_id(0)
    xb = x_ref[...].astype(jnp.bfloat16)
    sel = jax.lax.dot_general(
        xb, comb_ref[...],
        dimension_numbers=(((2,), (0,)), ((), ())),
        preferred_element_type=jnp.float32)
    l0 = sel[:, :, 0:_L]
    l1 = sel[:, :, _L:2 * _L]
    l2 = sel[:, :, 2 * _L:3 * _L]
    w_f = w_ref[...].astype(jnp.float32)
    p0 = l0 * w_f[0][None]
    p1 = l1 * w_f[1][None]
    p2 = l2 * w_f[2][None]
    sum_a = (p0 + p2) + p1
    sum_b = (p1 + p2) + p0

    hmat = (jax.lax.broadcasted_iota(jnp.int32, (_R, _K, _L), 1) * _L
            + jax.lax.broadcasted_iota(jnp.int32, (_R, _K, _L), 2))
    masks = [(hmat >= s) & (hmat < e) for s, e in zip(_STARTS, _ENDS)]
    logits = jnp.where(masks[2], sum_b, sum_a) + bias_ref[...]

    def _seg_reduce(arr, red, comb):
        r = lambda a: red(a, axis=(1, 2), keepdims=True)
        s0 = comb(r(arr[:, :68, :]), r(arr[:, 68:69, :64]))
        s1 = comb(r(arr[:, 68:69, 64:]), r(arr[:, 69:131, :]))
        s2 = comb(r(arr[:, 131:193, :]), r(arr[:, 193:194, :64]))
        s3 = comb(r(arr[:, 193:194, 64:]), r(arr[:, 194:, :]))
        return [s0, s1, s2, s3]

    mseg = _seg_reduce(logits, jnp.max, jnp.maximum)
    mfull = jnp.where(masks[0], mseg[0],
                      jnp.where(masks[1], mseg[1],
                                jnp.where(masks[2], mseg[2], mseg[3])))
    ez = jnp.exp(logits - mfull)
    sseg = _seg_reduce(ez, jnp.sum, jnp.add)
    sfull = jnp.where(masks[0], sseg[0],
                      jnp.where(masks[1], sseg[1],
                                jnp.where(masks[2], sseg[2], sseg[3])))
    p = ez / sfull
    logp = jnp.log(p)
    best = _first_argmax(p, hmat)

    k0 = key_ref[0].astype(jnp.uint32)
    k1 = key_ref[1].astype(jnp.uint32)
    row = (i * _R + jax.lax.broadcasted_iota(jnp.int32, (_R, _K, _L), 0))
    base = row * _H + hmat
    samples = []
    for e in range(3):
        idx = (base + np.int32(e * _B * _H)).astype(jnp.uint32)
        v = logp + _gumbel(k0, k1, idx)
        samples.append(_first_argmax(v, hmat))
    s0, s1, s2 = samples
    chosen = jnp.where(s0 == best, s0, jnp.where(s1 == best, s1, s2))
    out_ref[0, 0, :] = chosen


def kernel(X, W, b):
    X3 = X.reshape(_B, _K, _C * _L)

    wbf = W.astype(jnp.bfloat16)
    wcols = []
    for c in range(_C):
        parts = [jnp.broadcast_to(wbf[s, c, 0], (_ENDS[s] - _STARTS[s],))
                 for s in range(4)]
        wcols.append(jnp.concatenate(parts))
    wfull = jnp.stack(wcols).reshape(_C, _K, _L)
    bparts = [jnp.broadcast_to(b[s, 0], (_ENDS[s] - _STARTS[s],)) for s in range(4)]
    bfull = jnp.concatenate(bparts).reshape(1, _K, _L)

    j = np.arange(_C * _L)[:, None]
    cd = np.arange(_C * _L)[None, :]
    comb = jnp.asarray(j == 3 * (cd % _L) + cd // _L, dtype=jnp.bfloat16)

    kd = jax.random.key_data(jax.random.split(jax.random.key(1))[1])
    kd = kd.astype(jnp.int32)

    grid = _B // _R
    out = pl.pallas_call(
        _body,
        grid=grid,
        in_specs=[
            pl.BlockSpec(memory_space=pltpu.SMEM),
            pl.BlockSpec((_R, _K, _C * _L), lambda i: (i, 0, 0)),
            pl.BlockSpec((_C, _K, _L), lambda i: (0, 0, 0)),
            pl.BlockSpec((1, _K, _L), lambda i: (0, 0, 0)),
            pl.BlockSpec((_C * _L, _C * _L), lambda i: (0, 0)),
        ],
        out_specs=pl.BlockSpec((1, 1, _R), lambda i: (i, 0, 0)),
        out_shape=jax.ShapeDtypeStruct((grid, 1, _R), jnp.int32),
        compiler_params=pltpu.CompilerParams(
            dimension_semantics=("parallel",)),
    )(kd, X3, wfull, bfull, comb)
    return out.reshape(_B)


# speedup vs baseline: 1.7014x; 1.0226x over previous
import numpy as np
import jax
import jax.numpy as jnp
from jax.experimental import pallas as pl
from jax.experimental.pallas import tpu as pltpu

_B = 1024
_H = 32768
_C = 3
_R = 8
_K = 256
_L = 128
_W3 = _K * _L * _C // _K
_STARTS = (0, 8768, 16768, 24768)
_ENDS = (8768, 16768, 24768, 32768)
_TINY = np.float32(np.finfo(np.float32).tiny)
_NEG_INF = np.float32(-np.inf)


def _threefry_bits(k0, k1, x1):
    rot_a = (13, 15, 26, 6)
    rot_b = (17, 29, 16, 24)
    ks2 = k0 ^ k1 ^ np.uint32(0x1BD11BDA)
    x0 = jnp.broadcast_to(k0, x1.shape)
    x1 = x1 + k1

    def rounds(x0, x1, rots):
        for r in rots:
            x0 = x0 + x1
            x1 = (x1 << np.uint32(r)) | (x1 >> np.uint32(32 - r))
            x1 = x0 ^ x1
        return x0, x1

    x0, x1 = rounds(x0, x1, rot_a)
    x0, x1 = x0 + k1, x1 + ks2 + np.uint32(1)
    x0, x1 = rounds(x0, x1, rot_b)
    x0, x1 = x0 + ks2, x1 + k0 + np.uint32(2)
    x0, x1 = rounds(x0, x1, rot_a)
    x0, x1 = x0 + k0, x1 + k1 + np.uint32(3)
    x0, x1 = rounds(x0, x1, rot_b)
    x0, x1 = x0 + k1, x1 + ks2 + np.uint32(4)
    x0, x1 = rounds(x0, x1, rot_a)
    x0, x1 = x0 + ks2, x1 + k0 + np.uint32(5)
    return x0 ^ x1


def _gumbel(k0, k1, idx):
    bits = _threefry_bits(k0, k1, idx)
    fb = (bits >> np.uint32(9)) | np.uint32(0x3F800000)
    floats = jax.lax.bitcast_convert_type(fb, jnp.float32) - np.float32(1.0)
    u = floats + _TINY
    return -jnp.log(-jnp.log(u))


def _first_argmax(v, hmat):
    m = jnp.max(v, axis=(1, 2), keepdims=True)
    sel = jnp.where(v == m, hmat, jnp.int32(_H))
    return jnp.min(sel, axis=(1, 2))


def _body(key_ref, x_ref, w_ref, bias_ref, comb_ref, out_ref):
    i = pl.program_id(0)
    xb = x_ref[...].astype(jnp.bfloat16)
    sel = jax.lax.dot_general(
        xb, comb_ref[...],
        dimension_numbers=(((2,), (0,)), ((), ())),
        preferred_element_type=jnp.float32)
    l0 = sel[:, :, 0:_L]
    l1 = sel[:, :, _L:2 * _L]
    l2 = sel[:, :, 2 * _L:3 * _L]
    w_f = w_ref[...].astype(jnp.float32)
    p0 = l0 * w_f[0][None]
    p1 = l1 * w_f[1][None]
    p2 = l2 * w_f[2][None]
    sum_a = (p0 + p2) + p1
    sl2 = slice(131, 194)
    sum_b = (p1[:, sl2] + p2[:, sl2]) + p0[:, sl2]
    logits = jnp.concatenate([
        sum_a[:, :131, :],
        sum_b[:, :62, :],
        jnp.concatenate([sum_b[:, 62:63, :64], sum_a[:, 193:194, 64:]], axis=2),
        sum_a[:, 194:, :],
    ], axis=1) + bias_ref[...]

    def _seg_reduce(arr, red, comb):
        r = lambda a: red(a, axis=(1, 2), keepdims=True)
        s0 = comb(r(arr[:, :68, :]), r(arr[:, 68:69, :64]))
        s1 = comb(r(arr[:, 68:69, 64:]), r(arr[:, 69:131, :]))
        s2 = comb(r(arr[:, 131:193, :]), r(arr[:, 193:194, :64]))
        s3 = comb(r(arr[:, 193:194, 64:]), r(arr[:, 194:, :]))
        return [s0, s1, s2, s3]

    def _seg_field(s0, s1, s2, s3):
        bc = jnp.broadcast_to
        mix = lambda a, b: jnp.concatenate(
            [bc(a, (_R, 1, 64)), bc(b, (_R, 1, 64))], axis=2)
        return jnp.concatenate([
            bc(s0, (_R, 68, _L)), mix(s0, s1),
            bc(s1, (_R, 62, _L)), bc(s2, (_R, 62, _L)),
            mix(s2, s3), bc(s3, (_R, 62, _L)),
        ], axis=1)

    mseg = _seg_reduce(logits, jnp.max, jnp.maximum)
    mfull = _seg_field(*mseg)
    ez = jnp.exp(logits - mfull)
    sseg = _seg_reduce(ez, jnp.sum, jnp.add)
    sfull = _seg_field(*sseg)
    p = ez / sfull
    hmat = (jax.lax.broadcasted_iota(jnp.int32, (_R, _K, _L), 1) * _L
            + jax.lax.broadcasted_iota(jnp.int32, (_R, _K, _L), 2))
    logp = jnp.log(p)
    best = _first_argmax(p, hmat)

    k0 = key_ref[0].astype(jnp.uint32)
    k1 = key_ref[1].astype(jnp.uint32)
    row = (i * _R + jax.lax.broadcasted_iota(jnp.int32, (_R, _K, _L), 0))
    base = row * _H + hmat
    samples = []
    for e in range(3):
        idx = (base + np.int32(e * _B * _H)).astype(jnp.uint32)
        v = logp + _gumbel(k0, k1, idx)
        samples.append(_first_argmax(v, hmat))
    s0, s1, s2 = samples
    chosen = jnp.where(s0 == best, s0, jnp.where(s1 == best, s1, s2))
    out_ref[0, 0, :] = chosen


def kernel(X, W, b):
    X3 = X.reshape(_B, _K, _C * _L)

    wbf = W.astype(jnp.bfloat16)
    wcols = []
    for c in range(_C):
        parts = [jnp.broadcast_to(wbf[s, c, 0], (_ENDS[s] - _STARTS[s],))
                 for s in range(4)]
        wcols.append(jnp.concatenate(parts))
    wfull = jnp.stack(wcols).reshape(_C, _K, _L)
    bparts = [jnp.broadcast_to(b[s, 0], (_ENDS[s] - _STARTS[s],)) for s in range(4)]
    bfull = jnp.concatenate(bparts).reshape(1, _K, _L)

    j = np.arange(_C * _L)[:, None]
    cd = np.arange(_C * _L)[None, :]
    comb = jnp.asarray(j == 3 * (cd % _L) + cd // _L, dtype=jnp.bfloat16)

    kd = jax.random.key_data(jax.random.split(jax.random.key(1))[1])
    kd = kd.astype(jnp.int32)

    grid = _B // _R
    out = pl.pallas_call(
        _body,
        grid=grid,
        in_specs=[
            pl.BlockSpec(memory_space=pltpu.SMEM),
            pl.BlockSpec((_R, _K, _C * _L), lambda i: (i, 0, 0)),
            pl.BlockSpec((_C, _K, _L), lambda i: (0, 0, 0)),
            pl.BlockSpec((1, _K, _L), lambda i: (0, 0, 0)),
            pl.BlockSpec((_C * _L, _C * _L), lambda i: (0, 0)),
        ],
        out_specs=pl.BlockSpec((1, 1, _R), lambda i: (i, 0, 0)),
        out_shape=jax.ShapeDtypeStruct((grid, 1, _R), jnp.int32),
        compiler_params=pltpu.CompilerParams(
            dimension_semantics=("parallel",)),
    )(kd, X3, wfull, bfull, comb)
    return out.reshape(_B)


# speedup vs baseline: 1.7094x; 1.0047x over previous
import numpy as np
import jax
import jax.numpy as jnp
from jax.experimental import pallas as pl
from jax.experimental.pallas import tpu as pltpu

_B = 1024
_H = 32768
_C = 3
_R = 8
_K = 256
_L = 128
_W3 = _K * _L * _C // _K
_STARTS = (0, 8768, 16768, 24768)
_ENDS = (8768, 16768, 24768, 32768)
_TINY = np.float32(np.finfo(np.float32).tiny)
_NEG_INF = np.float32(-np.inf)


def _threefry_bits(k0, k1, x1):
    rot_a = (13, 15, 26, 6)
    rot_b = (17, 29, 16, 24)
    ks2 = k0 ^ k1 ^ np.uint32(0x1BD11BDA)
    x0 = jnp.broadcast_to(k0, x1.shape)

    def rounds(x0, x1, rots):
        for r in rots:
            x0 = x0 + x1
            x1 = (x1 << np.uint32(r)) | (x1 >> np.uint32(32 - r))
            x1 = x0 ^ x1
        return x0, x1

    x0, x1 = rounds(x0, x1, rot_a)
    x0, x1 = x0 + k1, x1 + ks2 + np.uint32(1)
    x0, x1 = rounds(x0, x1, rot_b)
    x0, x1 = x0 + ks2, x1 + k0 + np.uint32(2)
    x0, x1 = rounds(x0, x1, rot_a)
    x0, x1 = x0 + k0, x1 + k1 + np.uint32(3)
    x0, x1 = rounds(x0, x1, rot_b)
    x0, x1 = x0 + k1, x1 + ks2 + np.uint32(4)
    x0, x1 = rounds(x0, x1, rot_a)
    x0, x1 = x0 + ks2, x1 + k0 + np.uint32(5)
    return x0 ^ x1


def _gumbel(k0, k1, idx):
    bits = _threefry_bits(k0, k1, idx)
    fb = (bits >> np.uint32(9)) | np.uint32(0x3F800000)
    floats = jax.lax.bitcast_convert_type(fb, jnp.float32) - np.float32(1.0)
    u = floats + _TINY
    return -jnp.log(-jnp.log(u))


def _first_argmax(v, hmat):
    m = jnp.max(v, axis=(1, 2), keepdims=True)
    sel = jnp.where(v == m, hmat, jnp.int32(_H))
    return jnp.min(sel, axis=(1, 2))


def _body(key_ref, x_ref, w_ref, bias_ref, comb_ref, out_ref):
    i = pl.program_id(0)
    xb = x_ref[...].astype(jnp.bfloat16)
    sel = jax.lax.dot_general(
        xb, comb_ref[...],
        dimension_numbers=(((2,), (0,)), ((), ())),
        preferred_element_type=jnp.float32)
    l0 = sel[:, :, 0:_L]
    l1 = sel[:, :, _L:2 * _L]
    l2 = sel[:, :, 2 * _L:3 * _L]
    w_f = w_ref[...].astype(jnp.float32)
    p0 = l0 * w_f[0][None]
    p1 = l1 * w_f[1][None]
    p2 = l2 * w_f[2][None]
    sum_a = (p0 + p2) + p1
    sl2 = slice(131, 194)
    sum_b = (p1[:, sl2] + p2[:, sl2]) + p0[:, sl2]
    logits = jnp.concatenate([
        sum_a[:, :131, :],
        sum_b[:, :62, :],
        jnp.concatenate([sum_b[:, 62:63, :64], sum_a[:, 193:194, 64:]], axis=2),
        sum_a[:, 194:, :],
    ], axis=1) + bias_ref[...]

    def _seg_reduce(arr, red, comb):
        r = lambda a: red(a, axis=(1, 2), keepdims=True)
        s0 = comb(r(arr[:, :68, :]), r(arr[:, 68:69, :64]))
        s1 = comb(r(arr[:, 68:69, 64:]), r(arr[:, 69:131, :]))
        s2 = comb(r(arr[:, 131:193, :]), r(arr[:, 193:194, :64]))
        s3 = comb(r(arr[:, 193:194, 64:]), r(arr[:, 194:, :]))
        return [s0, s1, s2, s3]

    def _seg_field(s0, s1, s2, s3):
        bc = jnp.broadcast_to
        mix = lambda a, b: jnp.concatenate(
            [bc(a, (_R, 1, 64)), bc(b, (_R, 1, 64))], axis=2)
        return jnp.concatenate([
            bc(s0, (_R, 68, _L)), mix(s0, s1),
            bc(s1, (_R, 62, _L)), bc(s2, (_R, 62, _L)),
            mix(s2, s3), bc(s3, (_R, 62, _L)),
        ], axis=1)

    mseg = _seg_reduce(logits, jnp.max, jnp.maximum)
    mfull = _seg_field(*mseg)
    ez = jnp.exp(logits - mfull)
    sseg = _seg_reduce(ez, jnp.sum, jnp.add)
    sfull = _seg_field(*sseg)
    p = ez / sfull
    hmat = (jax.lax.broadcasted_iota(jnp.int32, (_R, _K, _L), 1) * _L
            + jax.lax.broadcasted_iota(jnp.int32, (_R, _K, _L), 2))
    logp = jnp.log(p)
    best = _first_argmax(p, hmat)

    k0 = key_ref[0].astype(jnp.uint32)
    k1 = key_ref[1].astype(jnp.uint32)
    row = (i * _R + jax.lax.broadcasted_iota(jnp.int32, (_R, _K, _L), 0))
    base = (row * _H + hmat).astype(jnp.uint32) + k1
    samples = []
    for e in range(3):
        v = logp + _gumbel(k0, k1, base + np.uint32(e * _B * _H))
        samples.append(_first_argmax(v, hmat))
    s0, s1, s2 = samples
    chosen = jnp.where(s0 == best, s0, jnp.where(s1 == best, s1, s2))
    out_ref[0, 0, :] = chosen


def kernel(X, W, b):
    X3 = X.reshape(_B, _K, _C * _L)

    wbf = W.astype(jnp.bfloat16)
    wcols = []
    for c in range(_C):
        parts = [jnp.broadcast_to(wbf[s, c, 0], (_ENDS[s] - _STARTS[s],))
                 for s in range(4)]
        wcols.append(jnp.concatenate(parts))
    wfull = jnp.stack(wcols).reshape(_C, _K, _L)
    bparts = [jnp.broadcast_to(b[s, 0], (_ENDS[s] - _STARTS[s],)) for s in range(4)]
    bfull = jnp.concatenate(bparts).reshape(1, _K, _L)

    j = np.arange(_C * _L)[:, None]
    cd = np.arange(_C * _L)[None, :]
    comb = jnp.asarray(j == 3 * (cd % _L) + cd // _L, dtype=jnp.bfloat16)

    kd = jax.random.key_data(jax.random.split(jax.random.key(1))[1])
    kd = kd.astype(jnp.int32)

    grid = _B // _R
    out = pl.pallas_call(
        _body,
        grid=grid,
        in_specs=[
            pl.BlockSpec(memory_space=pltpu.SMEM),
            pl.BlockSpec((_R, _K, _C * _L), lambda i: (i, 0, 0)),
            pl.BlockSpec((_C, _K, _L), lambda i: (0, 0, 0)),
            pl.BlockSpec((1, _K, _L), lambda i: (0, 0, 0)),
            pl.BlockSpec((_C * _L, _C * _L), lambda i: (0, 0)),
        ],
        out_specs=pl.BlockSpec((1, 1, _R), lambda i: (i, 0, 0)),
        out_shape=jax.ShapeDtypeStruct((grid, 1, _R), jnp.int32),
        compiler_params=pltpu.CompilerParams(
            dimension_semantics=("parallel",)),
    )(kd, X3, wfull, bfull, comb)
    return out.reshape(_B)
